# baseline (speedup 1.0000x reference)
import functools

import jax
import jax.numpy as jnp
from jax import lax
from jax.experimental import pallas as pl
from jax.experimental.pallas import tpu as pltpu
from jax.experimental.pallas import tpu_sc as plsc

_LSV_INDEX = 0
_SCALE = 1.0

_D = 2048
_NC = 2
_NS = 16
_NW = _NC * _NS
_CH = 8
_L = 16

_ROWS_SC = 8192


def _tc_add_body(x_ref, m_ref, o_ref):
    o_ref[...] = x_ref[...] + m_ref[_LSV_INDEX % 8, :] * _SCALE


def _tc_add(x2, lsv_matrix, rows_tc):
    rows = x2.shape[0]
    blk = 1024
    return pl.pallas_call(
        _tc_add_body,
        grid=(rows_tc // blk,),
        in_specs=[
            pl.BlockSpec((blk, _D), lambda i: (i, 0)),
            pl.BlockSpec((8, _D), lambda i: (_LSV_INDEX // 8, 0)),
        ],
        out_specs=pl.BlockSpec((blk, _D), lambda i: (i, 0)),
        out_shape=jax.ShapeDtypeStruct((rows, _D), x2.dtype),
    )(x2, lsv_matrix)


_SKIP_COMPUTE = False
_NBUF = 4
_JB = 16


def _make_sc_add(rows, row_offset):
    rows_pw = rows // _NW
    nch = rows_pw // _CH
    assert nch % _NBUF == 0
    mesh = plsc.VectorSubcoreMesh(core_axis_name="c", subcore_axis_name="s")

    @functools.partial(
        pl.kernel,
        mesh=mesh,
        out_type=jax.ShapeDtypeStruct((rows, _D), jnp.float32),
        scratch_types=[pltpu.VMEM((1, _D), jnp.float32)]
        + [pltpu.VMEM((_CH, _D), jnp.float32)] * _NBUF
        + [pltpu.SemaphoreType.DMA] * (2 * _NBUF),
    )
    def sc_add(x_hbm, m_hbm, out_hbm, vbuf, *rest):
        bufs = rest[:_NBUF]
        isems = rest[_NBUF : 2 * _NBUF]
        osems = rest[2 * _NBUF :]
        wid = lax.axis_index("s") * _NC + lax.axis_index("c")
        pltpu.sync_copy(m_hbm.at[pl.ds(_LSV_INDEX, 1)], vbuf)
        base0 = wid * rows_pw

        def in_copy(b, c):
            return pltpu.make_async_copy(
                x_hbm.at[pl.ds(row_offset + base0 + c * _CH, _CH)],
                bufs[b],
                isems[b],
            )

        def out_copy(b, c):
            return pltpu.make_async_copy(
                bufs[b], out_hbm.at[pl.ds(base0 + c * _CH, _CH)], osems[b]
            )

        def compute(b):
            buf = bufs[b]
            for jb in range(_D // _L // _JB):
                vs = [
                    vbuf[0, pl.ds((jb * _JB + t) * _L, _L)] for t in range(_JB)
                ]

                def rbody(r, cr, jb=jb, vs=vs, buf=buf):
                    for t in range(_JB):
                        sl = pl.ds((jb * _JB + t) * _L, _L)
                        buf[r, sl] = buf[r, sl] + vs[t]
                    return cr

                lax.fori_loop(0, _CH, rbody, 0, unroll=False)

        in_copy(0, 0).start()
        in_copy(1, 1).start()

        def group(g, carry):
            for k in range(_NBUF):
                c = g * _NBUF + k

                @pl.when(c >= 2)
                def _(c=c, k=k):
                    b2 = (k + 2) % _NBUF
                    out_copy(b2, c - 2).wait()

                @pl.when(c + 2 < nch)
                def _(c=c, k=k):
                    b2 = (k + 2) % _NBUF
                    in_copy(b2, c + 2).start()

                in_copy(k, c).wait()
                if not _SKIP_COMPUTE:
                    compute(k)
                out_copy(k, c).start()
            return carry

        lax.fori_loop(0, nch // _NBUF, group, 0, unroll=False)
        out_copy((nch - 2) % _NBUF, nch - 2).wait()
        out_copy((nch - 1) % _NBUF, nch - 1).wait()

    return sc_add


def kernel(x, lsv_matrix):
    b, s, d = x.shape
    rows = b * s
    x2 = x.reshape(rows, d)
    m = lsv_matrix if _SCALE == 1.0 else lsv_matrix * _SCALE

    rows_tc = rows - _ROWS_SC
    if _ROWS_SC == 0:
        out = _tc_add(x2, m, rows)
    elif rows_tc == 0:
        out = _make_sc_add(rows, 0)(x2, m)
    else:
        base_full = _tc_add(x2, m, rows_tc)
        sc_part = _make_sc_add(_ROWS_SC, rows_tc)(x2, m)
        out = lax.dynamic_update_slice(base_full, sc_part, (rows_tc, 0))
    return out.reshape(b, s, d)


# speedup vs baseline: 1.3702x; 1.3702x over previous
import jax
import jax.numpy as jnp
from jax.experimental import pallas as pl

_LSV_INDEX = 0
_SCALE = 1.0
_D = 2048
_BLK = 1024


def _tc_add_body(x_ref, m_ref, o_ref):
    o_ref[...] = x_ref[...] + m_ref[_LSV_INDEX % 8, :] * _SCALE


def kernel(x, lsv_matrix):
    b, s, d = x.shape
    rows = b * s
    x2 = x.reshape(rows, d)
    out = pl.pallas_call(
        _tc_add_body,
        grid=(rows // _BLK,),
        in_specs=[
            pl.BlockSpec((_BLK, _D), lambda i: (i, 0)),
            pl.BlockSpec((8, _D), lambda i: (_LSV_INDEX // 8, 0)),
        ],
        out_specs=pl.BlockSpec((_BLK, _D), lambda i: (i, 0)),
        out_shape=jax.ShapeDtypeStruct((rows, _D), x.dtype),
    )(x2, lsv_matrix)
    return out.reshape(b, s, d)
